# Initial kernel scaffold; baseline (speedup 1.0000x reference)
import jax
import jax.numpy as jnp
from jax.experimental import pallas as pl


def kernel(x, edge_index, gx, W1, b1, Wdr, bdr, Wg1, bg1, Wg2, bg2, W2, b2, W3, b3, Wout, bout):
    raise NotImplementedError("write your pallas kernel here")


# speedup vs baseline: 7.1519x; 7.1519x over previous
import functools

import jax
import jax.numpy as jnp
from jax import lax
from jax.experimental import pallas as pl
from jax.experimental.pallas import tpu as pltpu
from jax.experimental.pallas import tpu_sc as plsc

_N = 10000
_NP = 10240
_D = 128
_E = 320000
_NC, _NS = 2, 16
_NW = _NC * _NS
_EPT = 10240
_CHUNK = 128
_NCHUNKS = _EPT // _CHUNK
_RPS = _NP // _NS
_PAD_DST = 10016
_DW = 16


def _sc_mesh():
    return plsc.VectorSubcoreMesh(core_axis_name="c", subcore_axis_name="s")


def _sc_degree(dstp):
    @functools.partial(
        pl.kernel,
        mesh=_sc_mesh(),
        out_type=jax.ShapeDtypeStruct((_NC, _NP), jnp.float32),
        scratch_types=[
            pltpu.VMEM((_NCHUNKS, _CHUNK), jnp.int32),
            pltpu.VMEM((_CHUNK,), jnp.float32),
            pltpu.VMEM_SHARED((_NP,), jnp.float32),
        ],
    )
    def k(dst_hbm, out_hbm, dst_v, ones_v, deg_sh):
        cid = lax.axis_index("c")
        sid = lax.axis_index("s")
        wid = sid * _NC + cid
        pltpu.sync_copy(dst_hbm.at[wid], dst_v)

        ones16 = jnp.ones((16,), jnp.float32)
        zeros16 = jnp.zeros((16,), jnp.float32)

        def _z(r, c):
            ones_v[pl.ds(r * 16, 16)] = zeros16
            return c

        lax.fori_loop(0, _CHUNK // 16, _z, 0)
        base = sid * _RPS
        for t in range(_RPS // _CHUNK):
            pltpu.sync_copy(ones_v, deg_sh.at[pl.ds(base + t * _CHUNK, _CHUNK)])

        def _o(r, c):
            ones_v[pl.ds(r * 16, 16)] = ones16
            return c

        lax.fori_loop(0, _CHUNK // 16, _o, 0)
        plsc.subcore_barrier()

        def _step(j, c):
            pltpu.sync_copy(ones_v, deg_sh.at[dst_v.at[j]], add=True)
            return c

        lax.fori_loop(0, _NCHUNKS, _step, 0)
        plsc.subcore_barrier()
        pltpu.sync_copy(deg_sh.at[pl.ds(base, _RPS)],
                        out_hbm.at[cid, pl.ds(base, _RPS)])

    return k(dstp)


def _sc_scatter(h_tab, srcp, dstp):
    @functools.partial(
        pl.kernel,
        mesh=_sc_mesh(),
        out_type=jax.ShapeDtypeStruct((_NC, _NP, _D), jnp.float32),
        scratch_types=[
            pltpu.VMEM((_NCHUNKS, _CHUNK), jnp.int32),
            pltpu.VMEM((_NCHUNKS, _CHUNK), jnp.int32),
            pltpu.VMEM((_CHUNK, _D), jnp.float32),
            pltpu.SemaphoreType.DMA,
            pltpu.VMEM_SHARED((_NP, _D), jnp.float32),
        ],
    )
    def k(h_hbm, src_hbm, dst_hbm, out_hbm, src_v, dst_v, rows_v, sem,
          acc_sh):
        cid = lax.axis_index("c")
        sid = lax.axis_index("s")
        wid = sid * _NC + cid
        pltpu.sync_copy(src_hbm.at[wid], src_v)
        pltpu.sync_copy(dst_hbm.at[wid], dst_v)

        zeros16 = jnp.zeros((16,), jnp.float32)

        def _z(r, c):
            for q in range(_D // 16):
                rows_v[r, pl.ds(q * 16, 16)] = zeros16
            return c

        lax.fori_loop(0, _CHUNK, _z, 0)

        base = sid * _RPS
        for t in range(_RPS // _CHUNK):
            pltpu.sync_copy(rows_v, acc_sh.at[pl.ds(base + t * _CHUNK, _CHUNK)])
        plsc.subcore_barrier()

        def _step(j, c):
            pltpu.async_copy(h_hbm.at[src_v.at[j]], rows_v, sem).wait()
            pltpu.sync_copy(rows_v, acc_sh.at[dst_v.at[j]], add=True)
            return c

        lax.fori_loop(0, _NCHUNKS, _step, 0)
        plsc.subcore_barrier()
        pltpu.sync_copy(acc_sh.at[pl.ds(base, _RPS)],
                        out_hbm.at[cid, pl.ds(base, _RPS)])

    return k(h_tab, srcp, dstp)


_R = 2048
_G = _NP // _R


def _dot(a, b):
    return jnp.dot(a, b, preferred_element_type=jnp.float32)


def _row_spec(width=_D):
    return pl.BlockSpec((_R, width), lambda i: (i, 0))


def _full_spec(shape):
    nd = len(shape)
    return pl.BlockSpec(shape, lambda i: (0,) * nd)


def _tc0(xp, gxp, w1a, w1b, b1, wdr, bdr, wg1):
    def body(x_r, gx_r, w1a_r, w1b_r, b1_r, wdr_r, bdr_r, wg1_r,
             z_r, z0_r, h1_r):
        gg = gx_r[...]
        z = jnp.maximum(
            _dot(x_r[...], w1a_r[...]) + _dot(gg, w1b_r[...]) + b1_r[...], 0.0)
        z_r[...] = z
        z0_r[...] = _dot(z, wdr_r[...]) + bdr_r[...]
        h1_r[...] = _dot(z + gg, wg1_r[...])

    out = jax.ShapeDtypeStruct((_NP, _D), jnp.float32)
    return pl.pallas_call(
        body,
        grid=(_G,),
        in_specs=[_row_spec(), _row_spec(),
                  _full_spec((_D, _D)), _full_spec((_D, _D)),
                  _full_spec((1, _D)),
                  _full_spec((_D, _D)), _full_spec((1, _D)),
                  _full_spec((_D, _D))],
        out_specs=[_row_spec(), _row_spec(), _row_spec()],
        out_shape=[out, out, out],
    )(xp, gxp, w1a, w1b, b1, wdr, bdr, wg1)


def _tc1(d0, d1, h1):
    def body(d0_r, d1_r, h1_r, hp_r, dinv_r):
        deg = d0_r[...] + d1_r[...] + 1.0
        dinv = lax.rsqrt(deg)
        dinv_r[...] = dinv
        hp_r[...] = dinv * h1_r[...]

    return pl.pallas_call(
        body,
        grid=(_G,),
        in_specs=[_row_spec(1), _row_spec(1), _row_spec()],
        out_specs=[_row_spec(), _row_spec(1)],
        out_shape=[jax.ShapeDtypeStruct((_NP, _D), jnp.float32),
                   jax.ShapeDtypeStruct((_NP, 1), jnp.float32)],
    )(d0, d1, h1)


def _tc2(s0, s1, hp, dinv, bg1, wg2):
    def body(s0_r, s1_r, hp_r, dinv_r, bg1_r, wg2_r, z1_r, h2p_r):
        dv = dinv_r[...]
        z1 = jnp.maximum(dv * (s0_r[...] + s1_r[...] + hp_r[...]) + bg1_r[...],
                         0.0)
        z1_r[...] = z1
        h2p_r[...] = dv * _dot(z1, wg2_r[...])

    out = jax.ShapeDtypeStruct((_NP, _D), jnp.float32)
    return pl.pallas_call(
        body,
        grid=(_G,),
        in_specs=[_row_spec(), _row_spec(), _row_spec(), _row_spec(1),
                  _full_spec((1, _D)), _full_spec((_D, _D))],
        out_specs=[_row_spec(), _row_spec()],
        out_shape=[out, out],
    )(s0, s1, hp, dinv, bg1, wg2)


def _tc3(t0, t1, h2p, dinv, bg2, z, z1, z0, w2a, w2b, w2c, b2, w3, b3, wo, bo):
    def body(t0_r, t1_r, h2p_r, dinv_r, bg2_r, z_r, z1_r, z0_r,
             w2a_r, w2b_r, w2c_r, b2_r, w3_r, b3_r, wo_r, bo_r, o_r):
        dv = dinv_r[...]
        z2 = jnp.maximum(
            dv * (t0_r[...] + t1_r[...] + h2p_r[...]) + bg2_r[...], 0.0)
        z3 = jnp.maximum(
            _dot(z_r[...], w2a_r[...]) + _dot(z1_r[...], w2b_r[...])
            + _dot(z2, w2c_r[...]) + b2_r[...], 0.0)
        z4 = jnp.maximum(_dot(z3 + z0_r[...], w3_r[...]) + b3_r[...], 0.0)
        o_r[...] = _dot(z4, wo_r[...]) + bo_r[...]

    return pl.pallas_call(
        body,
        grid=(_G,),
        in_specs=[_row_spec(), _row_spec(), _row_spec(), _row_spec(1),
                  _full_spec((1, _D)),
                  _row_spec(), _row_spec(), _row_spec(),
                  _full_spec((_D, _D)), _full_spec((_D, _D)),
                  _full_spec((_D, _D)), _full_spec((1, _D)),
                  _full_spec((_D, _D)), _full_spec((1, _D)),
                  _full_spec((_D, _D)), _full_spec((1, _D))],
        out_specs=[_row_spec()],
        out_shape=[jax.ShapeDtypeStruct((_NP, _D), jnp.float32)],
    )(t0, t1, h2p, dinv, bg2, z, z1, z0, w2a, w2b, w2c, b2, w3, b3, wo, bo)[0]


def kernel(x, edge_index, gx, W1, b1, Wdr, bdr, Wg1, bg1, Wg2, bg2, W2, b2,
           W3, b3, Wout, bout):
    xp = jnp.pad(x, ((0, _NP - _N), (0, 0)))
    gxp = jnp.pad(gx, ((0, _NP - _N), (0, 0)))
    pad_e = _NW * _EPT - _E
    srcp = jnp.concatenate(
        [edge_index[0], jnp.full((pad_e,), _N, jnp.int32)]
    ).reshape(_NW, _NCHUNKS, _CHUNK)
    dstp = jnp.concatenate(
        [edge_index[1], jnp.full((pad_e,), _PAD_DST, jnp.int32)]
    ).reshape(_NW, _NCHUNKS, _CHUNK)

    w1a, w1b = W1[:_D], W1[_D:]
    w2a, w2b, w2c = W2[:_D], W2[_D:2 * _D], W2[2 * _D:]
    wo = jnp.pad(Wout, ((0, 0), (0, _D - Wout.shape[1])))
    bo = jnp.pad(bout, ((0, _D - bout.shape[0]),)).reshape(1, _D)
    b1r = b1.reshape(1, _D)
    bdrr = bdr.reshape(1, _D)
    bg1r = bg1.reshape(1, _D)
    bg2r = bg2.reshape(1, _D)
    b2r = b2.reshape(1, _D)
    b3r = b3.reshape(1, _D)

    degp = _sc_degree(dstp)
    z, z0, h1 = _tc0(xp, gxp, w1a, w1b, b1r, Wdr, bdrr, Wg1)
    h1p, dinv = _tc1(degp[0].reshape(_NP, 1), degp[1].reshape(_NP, 1), h1)
    s = _sc_scatter(h1p, srcp, dstp)
    z1, h2p = _tc2(s[0], s[1], h1p, dinv, bg1r, Wg2)
    t = _sc_scatter(h2p, srcp, dstp)
    o = _tc3(t[0], t[1], h2p, dinv, bg2r, z, z1, z0, w2a, w2b, w2c, b2r,
             W3, b3r, wo, bo)
    return o[:_N, :40]
